# Initial kernel scaffold; baseline (speedup 1.0000x reference)
import jax
import jax.numpy as jnp
from jax.experimental import pallas as pl


def kernel(x, te):
    raise NotImplementedError("write your pallas kernel here")


# speedup vs baseline: 1.8644x; 1.8644x over previous
import functools

import jax
import jax.numpy as jnp
from jax.experimental import pallas as pl
from jax.experimental.pallas import tpu as pltpu

S, B, D = 4096, 4, 4096
N = S // 2
SBLK = 256
GRID = S // SBLK


def _gumbel_table():
    u = jax.random.uniform(jax.random.key(42), (B, S), minval=1e-20,
                           maxval=1.0)
    return -jnp.log(-jnp.log(u))


def _sortable_u32(f):
    b = jax.lax.bitcast_convert_type(f, jnp.int32)
    flip = jax.lax.shift_right_arithmetic(b, 31).astype(jnp.uint32) \
        | jnp.uint32(0x80000000)
    return b.astype(jnp.uint32) ^ flip


def _tc_body(x_ref, te_ref, g_ref, out_ref, w_acc):
    i = pl.program_id(0)

    xb = x_ref[...]
    te = te_ref[...]
    part = jnp.sum(xb * te[None, :, :], axis=-1)
    w_acc[:, pl.ds(i * SBLK, SBLK)] = part.T

    @pl.when(i == GRID - 1)
    def _():
        w = w_acc[...]
        g = g_ref[...]
        mx = jnp.max(w, axis=1, keepdims=True)
        t = mx - w
        tmx = jnp.max(t, axis=1, keepdims=True)
        p = jnp.exp(t - tmx)
        p_inv = p / jnp.sum(p, axis=1, keepdims=True)
        sc = jnp.log(p_inv + 1e-20) + g
        ku = _sortable_u32(sc)

        prefix = jnp.zeros((B, 1), jnp.uint32)
        for bit in range(31, -1, -1):
            cand = prefix | jnp.uint32(1 << bit)
            cnt = jnp.sum((ku >= cand).astype(jnp.int32), axis=1,
                          keepdims=True)
            prefix = jnp.where(cnt >= N, cand, prefix)
        thr = prefix

        gt = ku > thr
        eq = ku == thr
        r = N - jnp.sum(gt.astype(jnp.int32), axis=1, keepdims=True)
        idx = jax.lax.broadcasted_iota(jnp.int32, (B, S), 1)
        lo = jnp.zeros((B, 1), jnp.int32)
        hi = jnp.full((B, 1), S - 1, jnp.int32)
        for _ in range(12):
            mid = (lo + hi) >> 1
            cnt = jnp.sum((eq & (idx <= mid)).astype(jnp.int32), axis=1,
                          keepdims=True)
            take = cnt >= r
            hi = jnp.where(take, mid, hi)
            lo = jnp.where(take, lo, mid + 1)
        mask = gt | (eq & (idx <= hi) & (r > 0))

        neg = jnp.float32(-jnp.inf)
        m2 = jnp.max(jnp.where(mask, neg, w), axis=1, keepdims=True)
        e = jnp.where(mask, 0.0, jnp.exp(w - m2))
        out = e / jnp.sum(e, axis=1, keepdims=True)
        out_ref[...] = out.T


@jax.jit
def kernel(x, te):
    te2 = te[..., 0]
    g = _gumbel_table()
    out = pl.pallas_call(
        _tc_body,
        grid=(GRID,),
        in_specs=[
            pl.BlockSpec((SBLK, B, D), lambda i: (i, 0, 0)),
            pl.BlockSpec((B, D), lambda i: (0, 0)),
            pl.BlockSpec((B, S), lambda i: (0, 0)),
        ],
        out_specs=pl.BlockSpec((S, B), lambda i: (0, 0)),
        out_shape=jax.ShapeDtypeStruct((S, B), jnp.float32),
        scratch_shapes=[pltpu.VMEM((B, S), jnp.float32)],
        compiler_params=pltpu.CompilerParams(
            dimension_semantics=("arbitrary",),
        ),
    )(x, te2, g)
    return out[..., None]
